# Initial kernel scaffold; baseline (speedup 1.0000x reference)
import jax
import jax.numpy as jnp
from jax.experimental import pallas as pl


def kernel(hidden_states, w_gate, w_u, w_g, w_d):
    raise NotImplementedError("write your pallas kernel here")


# speedup vs baseline: 1.1456x; 1.1456x over previous
import jax
import jax.numpy as jnp
from jax.experimental import pallas as pl
from jax.experimental.pallas import tpu as pltpu

T = 2048
D = 768
F = 2048
E = 8
TOP_K = 2

RT = 512
BT = 256
EPAD = 128


def _router_body(x_ref, wgt_ref, comb_ref, sel_ref, topw_ref, aux_ref, acc_ref):
    i = pl.program_id(0)
    nsteps = pl.num_programs(0)
    logits = jnp.dot(x_ref[...], wgt_ref[...], preferred_element_type=jnp.float32)
    lane = jax.lax.broadcasted_iota(jnp.int32, logits.shape, 1)
    lm = jnp.where(lane < E, logits, -jnp.inf)
    m = jnp.max(lm, axis=1, keepdims=True)
    p = jnp.exp(lm - m)
    s = jnp.sum(p, axis=1, keepdims=True)
    z = jnp.log(s) + m

    @pl.when(i == 0)
    def _():
        acc_ref[0, 0] = 0.0

    acc_ref[0, 0] += jnp.sum(z * z)

    probs = p / s
    big = jnp.int32(999)
    p1 = jnp.max(probs, axis=1, keepdims=True)
    a1 = jnp.min(jnp.where(probs == p1, lane, big), axis=1, keepdims=True)
    probs2 = jnp.where(lane == a1, -1.0, probs)
    p2 = jnp.max(probs2, axis=1, keepdims=True)
    a2 = jnp.min(jnp.where(probs2 == p2, lane, big), axis=1, keepdims=True)
    wsum = p1 + p2
    w1 = p1 / wsum
    w2 = p2 / wsum
    comb = jnp.where(lane == a1, w1, 0.0) + jnp.where(lane == a2, w2, 0.0)
    comb_ref[...] = comb[:, :E]
    sel = jnp.where(lane == 0, a1, jnp.where(lane == 1, a2, 0))
    sel_ref[...] = sel[:, :TOP_K]
    topw = jnp.where(lane == 0, w1, jnp.where(lane == 1, w2, 0.0))
    topw_ref[...] = topw[:, :TOP_K]

    @pl.when(i == nsteps - 1)
    def _():
        aux_ref[0, 0] = acc_ref[0, 0] * (0.001 / T)


def _router(x, w_gate):
    wgt = jnp.zeros((D, EPAD), jnp.float32).at[:, :E].set(w_gate.T)
    return pl.pallas_call(
        _router_body,
        grid=(T // RT,),
        in_specs=[
            pl.BlockSpec((RT, D), lambda i: (i, 0)),
            pl.BlockSpec((D, EPAD), lambda i: (0, 0)),
        ],
        out_specs=[
            pl.BlockSpec((RT, E), lambda i: (i, 0)),
            pl.BlockSpec((RT, TOP_K), lambda i: (i, 0)),
            pl.BlockSpec((RT, TOP_K), lambda i: (i, 0)),
            pl.BlockSpec((1, 1), lambda i: (0, 0), memory_space=pltpu.SMEM),
        ],
        out_shape=[
            jax.ShapeDtypeStruct((T, E), jnp.float32),
            jax.ShapeDtypeStruct((T, TOP_K), jnp.int32),
            jax.ShapeDtypeStruct((T, TOP_K), jnp.float32),
            jax.ShapeDtypeStruct((1, 1), jnp.float32),
        ],
        scratch_shapes=[pltpu.SMEM((1, 1), jnp.float32)],
    )(x, wgt)


def _ffn_body(comb_ref, x_ref, wg_ref, wu_ref, wd_ref, out_ref, acc_ref):
    e = pl.program_id(0)
    tm = pl.program_id(1)
    x = x_ref[...]
    g = jnp.dot(x, wg_ref[0], preferred_element_type=jnp.float32)
    u = jnp.dot(x, wu_ref[0], preferred_element_type=jnp.float32)
    h = (g * jax.nn.sigmoid(g)) * u
    dn = jnp.dot(h, wd_ref[0], preferred_element_type=jnp.float32)
    lane = jax.lax.broadcasted_iota(jnp.int32, (BT, E), 1)
    w = jnp.sum(jnp.where(lane == e, comb_ref[...], 0.0), axis=1, keepdims=True)
    contrib = w * dn
    sl = pl.ds(tm * BT, BT)

    @pl.when(e == 0)
    def _():
        acc_ref[sl, :] = contrib

    @pl.when(e > 0)
    def _():
        acc_ref[sl, :] += contrib

    @pl.when(e == E - 1)
    def _():
        out_ref[...] = acc_ref[sl, :]


def _ffn(comb, x, gate_w, up_w, down_w):
    return pl.pallas_call(
        _ffn_body,
        grid=(E, T // BT),
        in_specs=[
            pl.BlockSpec((BT, E), lambda e, tm: (tm, 0)),
            pl.BlockSpec((BT, D), lambda e, tm: (tm, 0)),
            pl.BlockSpec((1, D, F), lambda e, tm: (e, 0, 0)),
            pl.BlockSpec((1, D, F), lambda e, tm: (e, 0, 0)),
            pl.BlockSpec((1, F, D), lambda e, tm: (e, 0, 0)),
        ],
        out_specs=pl.BlockSpec((BT, D), lambda e, tm: (tm, 0)),
        out_shape=jax.ShapeDtypeStruct((T, D), jnp.float32),
        scratch_shapes=[pltpu.VMEM((T, D), jnp.float32)],
        compiler_params=pltpu.CompilerParams(
            dimension_semantics=("arbitrary", "arbitrary"),
            vmem_limit_bytes=110 * 1024 * 1024,
        ),
    )(comb, x, gate_w, up_w, down_w)


def kernel(hidden_states, w_gate, w_u, w_g, w_d):
    gate_w, up_w, down_w = w_u, w_g, w_d
    b, s, d = hidden_states.shape
    x = hidden_states.reshape(-1, d)
    comb, sel, topw, aux = _router(x, w_gate)
    final = _ffn(comb, x, gate_w, up_w, down_w)
    return final.reshape(b, s, d), aux.reshape(())
